# baseline (speedup 1.0000x reference)
import functools

import jax
import jax.numpy as jnp
from jax import lax
from jax.experimental import pallas as pl
from jax.experimental.pallas import tpu as pltpu
from jax.experimental.pallas import tpu_sc as plsc

NTOK = 8192
TOPK = 2
HID = 2048
NC = 2
NS = 16
NW = NC * NS
T = 8
ROWS = T * TOPK
CPW = NTOK // (NW * T)
LANES = 16
HREG = HID // LANES
NBUF = 3


@functools.partial(
    pl.kernel,
    out_type=jax.ShapeDtypeStruct((NTOK, HID), jnp.float32),
    mesh=plsc.VectorSubcoreMesh(core_axis_name="c", subcore_axis_name="s"),
    scratch_types=[
        pltpu.VMEM((CPW * ROWS,), jnp.int32),
        pltpu.VMEM((CPW * ROWS,), jnp.int32),
        pltpu.VMEM((NBUF, ROWS, HID), jnp.float32),
        pltpu.SemaphoreType.DMA,
        pltpu.SemaphoreType.DMA,
        pltpu.SemaphoreType.DMA,
    ],
)
def _gather_add(table_hbm, idx_hbm, out_hbm, idx_raw, idx_v, buf_v, sem0, sem1, sem2):
    wid = lax.axis_index("s") * NC + lax.axis_index("c")
    g0 = wid * CPW
    idx_base = pl.multiple_of(g0 * ROWS, 8)
    pltpu.sync_copy(idx_hbm.at[pl.ds(idx_base, CPW * ROWS)], idx_raw)

    lane = lax.broadcasted_iota(jnp.int32, (LANES,), 0)
    pattern = jnp.where(lane < T, 2 * lane, 2 * lane - (ROWS - 1))

    def perm_body(c, carry):
        base = pl.multiple_of(c * ROWS, 8)
        v = idx_raw[pl.ds(base, ROWS)]
        pv = lax.gather(
            v,
            pattern[:, None],
            dimension_numbers=lax.GatherDimensionNumbers(
                offset_dims=(), collapsed_slice_dims=(0,), start_index_map=(0,)
            ),
            slice_sizes=(1,),
            mode=lax.GatherScatterMode.PROMISE_IN_BOUNDS,
        )
        idx_v[pl.ds(base, ROWS)] = pv
        return carry

    lax.fori_loop(0, CPW, perm_body, 0)
    sems = [sem0, sem1, sem2]

    def start_gather(c_local, p):
        pltpu.make_async_copy(
            table_hbm.at[idx_v.at[pl.ds(c_local * ROWS, ROWS)]],
            buf_v.at[p],
            sems[p],
        ).start()

    def wait_gather(p):
        pltpu.make_async_copy(
            table_hbm.at[pl.ds(0, ROWS)],
            buf_v.at[p],
            sems[p],
        ).wait()

    for p in range(NBUF):
        start_gather(p, p)

    def chunk_step(c_local, p):
        wait_gather(p)

        def h_body(h, carry):
            base = pl.multiple_of(h * LANES, LANES)
            for t in range(T):
                v = buf_v[p, T + t, pl.ds(base, LANES)]
                plsc.addupdate(buf_v.at[p, t, pl.ds(base, LANES)], v)
            return carry

        lax.fori_loop(0, HREG, h_body, 0)

        row0 = pl.multiple_of((g0 + c_local) * T, 8)
        pltpu.sync_copy(buf_v.at[p, pl.ds(0, T)], out_hbm.at[pl.ds(row0, T)])


        if c_local + NBUF < CPW:
            start_gather(c_local + NBUF, p)

    for c in range(CPW):
        chunk_step(c, c % NBUF)


def kernel(gemm_buffer, outputs_buf, gemm_ready_flag, scatter_index, num_groups):
    return _gather_add(gemm_buffer, scatter_index)


# speedup vs baseline: 1.0530x; 1.0530x over previous
import functools

import jax
import jax.numpy as jnp
from jax import lax
from jax.experimental import pallas as pl
from jax.experimental.pallas import tpu as pltpu
from jax.experimental.pallas import tpu_sc as plsc

NTOK = 8192
TOPK = 2
HID = 2048
NC = 2
NS = 16
NW = NC * NS
T = 8
ROWS = T * TOPK
CPW = NTOK // (NW * T)
LANES = 16
HREG = HID // LANES
NBUF = 3


@functools.partial(
    pl.kernel,
    out_type=jax.ShapeDtypeStruct((NTOK, HID), jnp.float32),
    mesh=plsc.VectorSubcoreMesh(core_axis_name="c", subcore_axis_name="s"),
    scratch_types=[
        pltpu.VMEM((CPW * ROWS,), jnp.int32),
        pltpu.VMEM((CPW * ROWS,), jnp.int32),
        pltpu.VMEM((NBUF, ROWS, HID), jnp.float32),
        pltpu.SemaphoreType.DMA,
        pltpu.SemaphoreType.DMA,
        pltpu.SemaphoreType.DMA,
    ],
)
def _gather_add(table_hbm, idx_hbm, out_hbm, idx_raw, idx_v, buf_v, sem0, sem1, sem2):
    wid = lax.axis_index("s") * NC + lax.axis_index("c")
    g0 = wid * CPW
    idx_base = pl.multiple_of(g0 * ROWS, 8)
    pltpu.sync_copy(idx_hbm.at[pl.ds(idx_base, CPW * ROWS)], idx_raw)

    lane = lax.broadcasted_iota(jnp.int32, (LANES,), 0)
    pattern = jnp.where(lane < T, 2 * lane, 2 * lane - (ROWS - 1))

    def perm_body(c, carry):
        base = pl.multiple_of(c * ROWS, 8)
        v = idx_raw[pl.ds(base, ROWS)]
        pv = lax.gather(
            v,
            pattern[:, None],
            dimension_numbers=lax.GatherDimensionNumbers(
                offset_dims=(), collapsed_slice_dims=(0,), start_index_map=(0,)
            ),
            slice_sizes=(1,),
            mode=lax.GatherScatterMode.PROMISE_IN_BOUNDS,
        )
        idx_v[pl.ds(base, ROWS)] = pv
        return carry

    lax.fori_loop(0, CPW, perm_body, 0)
    sems = [sem0, sem1, sem2]

    def start_gather(c_local, p):
        pltpu.make_async_copy(
            table_hbm.at[idx_v.at[pl.ds(c_local * ROWS, ROWS)]],
            buf_v.at[p],
            sems[p],
        ).start()

    def wait_gather(p):
        pltpu.make_async_copy(
            table_hbm.at[pl.ds(0, ROWS)],
            buf_v.at[p],
            sems[p],
        ).wait()

    for p in range(NBUF):
        start_gather(p, p)

    def chunk_step(c_local, p, dynamic):
        wait_gather(p)

        def h_body(h, carry):
            base = pl.multiple_of(h * LANES, LANES)
            for t in range(T):
                v = buf_v[p, T + t, pl.ds(base, LANES)]
                plsc.addupdate(buf_v.at[p, t, pl.ds(base, LANES)], v)
            return carry

        lax.fori_loop(0, HREG, h_body, 0)

        row0 = pl.multiple_of((g0 + c_local) * T, 8)
        pltpu.sync_copy(buf_v.at[p, pl.ds(0, T)], out_hbm.at[pl.ds(row0, T)])

        if dynamic:
            @pl.when(c_local + NBUF < CPW)
            def _():
                start_gather(c_local + NBUF, p)
        elif c_local + NBUF < CPW:
            start_gather(c_local + NBUF, p)

    def loop_body(i, carry):
        for p in range(NBUF):
            chunk_step(NBUF * i + p, p, True)
        return carry

    lax.fori_loop(0, CPW // NBUF, loop_body, 0)
    for c in range(CPW - CPW % NBUF, CPW):
        chunk_step(c, c % NBUF, False)


def kernel(gemm_buffer, outputs_buf, gemm_ready_flag, scatter_index, num_groups):
    return _gather_add(gemm_buffer, scatter_index)
